# baseline (speedup 1.0000x reference)
import functools

import jax
import jax.numpy as jnp
from jax import lax
from jax.experimental import pallas as pl
from jax.experimental.pallas import tpu as pltpu
from jax.experimental.pallas import tpu_sc as plsc

N_E = 1024
E_DIM = 64
BETA = 0.25
LAMBDA_REG = 0.1
UNIFORM_WEIGHT = 0.1

N_ROWS = 18432
HALF = N_ROWS // 2
TILE = 2304

_NC = 2
_NS = 16
_NW = _NC * _NS
_BPW = HALF // _NW
_CHUNK = 96
_NCHUNK = _BPW // _CHUNK


def _argmin_qq_body(z_ref, emb_ref, idx_ref, qq_ref):
    i = pl.program_id(0)
    z = z_ref[...]
    e = emb_ref[...]

    zz = jnp.sum(z * z, axis=1, keepdims=True)
    ee = jnp.sum(e * e, axis=1)
    two_ze = 2.0 * jax.lax.dot_general(
        z, e, (((1,), (1,)), ((), ())), preferred_element_type=jnp.float32)
    d = (zz + ee[None, :]) - two_ze

    iota = jax.lax.broadcasted_iota(jnp.int32, (TILE, N_E), 1)
    dmin = jnp.min(d, axis=1, keepdims=True)
    idx = jnp.min(jnp.where(d == dmin, iota, N_E), axis=1)
    idx_ref[...] = idx[:, None]

    @pl.when(i == 0)
    def _qq():
        sq = (ee[:, None] + ee[None, :]) - 2.0 * jax.lax.dot_general(
            e, e, (((1,), (1,)), ((), ())), preferred_element_type=jnp.float32)
        sq = jnp.maximum(sq, 0.0)
        dist = jnp.where(sq > 0.0, jnp.sqrt(jnp.where(sq > 0.0, sq, 1.0)), 0.0)
        min_d = jnp.min(dist, axis=1)
        max_d = jnp.max(dist, axis=1)
        uniform_loss = jnp.mean(max_d - min_d)
        qq_ref[0, 0] = UNIFORM_WEIGHT * uniform_loss + LAMBDA_REG * jnp.sum(e * e)


def _argmin_body(z_ref, emb_ref, idx_ref):
    z = z_ref[...]
    e = emb_ref[...]
    zz = jnp.sum(z * z, axis=1, keepdims=True)
    ee = jnp.sum(e * e, axis=1)
    two_ze = 2.0 * jax.lax.dot_general(
        z, e, (((1,), (1,)), ((), ())), preferred_element_type=jnp.float32)
    d = (zz + ee[None, :]) - two_ze
    iota = jax.lax.broadcasted_iota(jnp.int32, (TILE, N_E), 1)
    dmin = jnp.min(d, axis=1, keepdims=True)
    idx = jnp.min(jnp.where(d == dmin, iota, N_E), axis=1)
    idx_ref[...] = idx[:, None]


def _vq_argmin(z_half, emb, with_qq):
    grid = HALF // TILE
    in_specs = [
        pl.BlockSpec((TILE, E_DIM), lambda i: (i, 0)),
        pl.BlockSpec((N_E, E_DIM), lambda i: (0, 0)),
    ]
    if with_qq:
        return pl.pallas_call(
            _argmin_qq_body,
            grid=(grid,),
            in_specs=in_specs,
            out_specs=[
                pl.BlockSpec((TILE, 1), lambda i: (i, 0)),
                pl.BlockSpec(memory_space=pltpu.SMEM),
            ],
            out_shape=[
                jax.ShapeDtypeStruct((HALF, 1), jnp.int32),
                jax.ShapeDtypeStruct((1, 1), jnp.float32),
            ],
            compiler_params=pltpu.CompilerParams(
                dimension_semantics=("arbitrary",)),
        )(z_half, emb)
    return pl.pallas_call(
        _argmin_body,
        grid=(grid,),
        in_specs=in_specs,
        out_specs=pl.BlockSpec((TILE, 1), lambda i: (i, 0)),
        out_shape=jax.ShapeDtypeStruct((HALF, 1), jnp.int32),
        compiler_params=pltpu.CompilerParams(
            dimension_semantics=("arbitrary",)),
    )(z_half, emb)


def _sc_gather(emb128, idx_flat):
    mesh = plsc.VectorSubcoreMesh(
        core_axis_name="c", subcore_axis_name="s",
        num_cores=_NC, num_subcores=_NS)

    @functools.partial(
        pl.kernel,
        mesh=mesh,
        out_type=jax.ShapeDtypeStruct((HALF, 128), jnp.float32),
        scratch_types=[
            pltpu.VMEM((_BPW,), jnp.int32),
            pltpu.VMEM((_BPW, 128), jnp.float32),
            pltpu.SemaphoreType.DMA,
        ],
    )
    def gather_k(table_hbm, idx_hbm, out_hbm, idx_v, rows_v, sem):
        wid = lax.axis_index("s") * _NC + lax.axis_index("c")
        base = wid * _BPW
        pltpu.sync_copy(idx_hbm.at[pl.ds(base, _BPW)], idx_v)
        copies = [
            pltpu.async_copy(
                table_hbm.at[idx_v.at[pl.ds(c * _CHUNK, _CHUNK)]],
                rows_v.at[pl.ds(c * _CHUNK, _CHUNK)],
                sem,
            )
            for c in range(_NCHUNK)
        ]
        for cp in copies:
            cp.wait()
        pltpu.sync_copy(rows_v, out_hbm.at[pl.ds(base, _BPW)])

    return gather_k(emb128, idx_flat)


def _finish_body(z_ref, zq_ref, out_ref, loss_ref):
    i = pl.program_id(0)
    z = z_ref[...]
    z_q = zq_ref[:, :E_DIM]
    out_ref[...] = z + (z_q - z)
    diff = z_q - z
    partial = jnp.sum(diff * diff)

    @pl.when(i == 0)
    def _init():
        loss_ref[0, 0] = partial

    @pl.when(i != 0)
    def _acc():
        loss_ref[0, 0] += partial


def _finish(z_half, z_q):
    grid = HALF // TILE
    zq_st, loss_sum = pl.pallas_call(
        _finish_body,
        grid=(grid,),
        in_specs=[
            pl.BlockSpec((TILE, E_DIM), lambda i: (i, 0)),
            pl.BlockSpec((TILE, 128), lambda i: (i, 0)),
        ],
        out_specs=[
            pl.BlockSpec((TILE, E_DIM), lambda i: (i, 0)),
            pl.BlockSpec(memory_space=pltpu.SMEM),
        ],
        out_shape=[
            jax.ShapeDtypeStruct((HALF, E_DIM), jnp.float32),
            jax.ShapeDtypeStruct((1, 1), jnp.float32),
        ],
        compiler_params=pltpu.CompilerParams(
            dimension_semantics=("arbitrary",)),
    )(z_half, z_q)
    return zq_st, loss_sum


def kernel(input, embedding_weight):
    z = input
    z_flat = z.reshape(-1, E_DIM)
    z1, z2 = z_flat[:HALF], z_flat[HALF:]
    emb128 = jnp.concatenate(
        [embedding_weight,
         jnp.zeros((N_E, 128 - E_DIM), jnp.float32)], axis=1)
    idx1, qq = _vq_argmin(z1, embedding_weight, True)
    zq1 = _sc_gather(emb128, idx1.reshape(-1))
    idx2 = _vq_argmin(z2, embedding_weight, False)
    zq2 = _sc_gather(emb128, idx2.reshape(-1))
    zq_st1, loss1 = _finish(z1, zq1)
    zq_st2, loss2 = _finish(z2, zq2)
    idx = jnp.concatenate([idx1, idx2], axis=0)
    zq_st = jnp.concatenate([zq_st1, zq_st2], axis=0)
    m = (loss1[0, 0] + loss2[0, 0]) / (N_ROWS * E_DIM)
    loss = m + BETA * m
    return (zq_st.reshape(z.shape), idx, loss, qq[0, 0])


# speedup vs baseline: 1.4566x; 1.4566x over previous
import functools

import jax
import jax.numpy as jnp
from jax.experimental import pallas as pl
from jax.experimental.pallas import tpu as pltpu

N_E = 1024
E_DIM = 64
BETA = 0.25
LAMBDA_REG = 0.1
UNIFORM_WEIGHT = 0.1

TILE = 3072


def _vq_body(z_ref, emb_ref, zq_ref, idx_ref, loss_ref, qq_ref):
    i = pl.program_id(0)
    z = z_ref[...]
    e = emb_ref[...]

    zz = jnp.sum(z * z, axis=1, keepdims=True)
    ee = jnp.sum(e * e, axis=1)
    two_ze = 2.0 * jax.lax.dot_general(
        z, e, (((1,), (1,)), ((), ())), preferred_element_type=jnp.float32)
    d = (zz + ee[None, :]) - two_ze

    iota = jax.lax.broadcasted_iota(jnp.int32, (TILE, N_E), 1)
    dmin = jnp.min(d, axis=1, keepdims=True)
    idx = jnp.min(jnp.where(d == dmin, iota, N_E), axis=1)
    idx_ref[...] = idx[:, None]

    one_hot = (iota == idx[:, None]).astype(jnp.float32)
    z_q = jax.lax.dot_general(
        one_hot, e, (((1,), (0,)), ((), ())), preferred_element_type=jnp.float32)
    zq_ref[...] = z + (z_q - z)

    diff = z_q - z
    partial = jnp.sum(diff * diff)

    @pl.when(i == 0)
    def _init():
        loss_ref[0, 0] = partial
        sq = (ee[:, None] + ee[None, :]) - 2.0 * jax.lax.dot_general(
            e, e, (((1,), (1,)), ((), ())), preferred_element_type=jnp.float32)
        sq = jnp.maximum(sq, 0.0)
        dist = jnp.where(sq > 0.0, jnp.sqrt(jnp.where(sq > 0.0, sq, 1.0)), 0.0)
        min_d = jnp.min(dist, axis=1)
        max_d = jnp.max(dist, axis=1)
        uniform_loss = jnp.mean(max_d - min_d)
        qq_ref[0, 0] = UNIFORM_WEIGHT * uniform_loss + LAMBDA_REG * jnp.sum(e * e)

    @pl.when(i != 0)
    def _acc():
        loss_ref[0, 0] += partial


@functools.partial(jax.jit, static_argnames=("interpret",))
def _vq_fused(z_flat, emb, interpret=False):
    n = z_flat.shape[0]
    grid = n // TILE
    zq, idx, loss_sum, qq = pl.pallas_call(
        _vq_body,
        grid=(grid,),
        in_specs=[
            pl.BlockSpec((TILE, E_DIM), lambda i: (i, 0)),
            pl.BlockSpec((N_E, E_DIM), lambda i: (0, 0)),
        ],
        out_specs=[
            pl.BlockSpec((TILE, E_DIM), lambda i: (i, 0)),
            pl.BlockSpec((TILE, 1), lambda i: (i, 0)),
            pl.BlockSpec(memory_space=pltpu.SMEM),
            pl.BlockSpec(memory_space=pltpu.SMEM),
        ],
        out_shape=[
            jax.ShapeDtypeStruct((n, E_DIM), jnp.float32),
            jax.ShapeDtypeStruct((n, 1), jnp.int32),
            jax.ShapeDtypeStruct((1, 1), jnp.float32),
            jax.ShapeDtypeStruct((1, 1), jnp.float32),
        ],
        compiler_params=pltpu.CompilerParams(
            dimension_semantics=("arbitrary",)),
        interpret=interpret,
    )(z_flat, emb)
    return zq, idx, loss_sum, qq


def kernel(input, embedding_weight):
    z = input
    z_flat = z.reshape(-1, E_DIM)
    zq, idx, loss_sum, qq = _vq_fused(z_flat, embedding_weight)
    m = loss_sum[0, 0] / (z_flat.shape[0] * E_DIM)
    loss = m + BETA * m
    return (zq.reshape(z.shape), idx, loss, qq[0, 0])


# speedup vs baseline: 1.4625x; 1.0040x over previous
import functools

import jax
import jax.numpy as jnp
from jax.experimental import pallas as pl
from jax.experimental.pallas import tpu as pltpu

N_E = 1024
E_DIM = 64
BETA = 0.25
LAMBDA_REG = 0.1
UNIFORM_WEIGHT = 0.1

TILE = 4608


def _vq_body(z_ref, emb_ref, zq_ref, idx_ref, loss_ref, qq_ref):
    i = pl.program_id(0)
    z = z_ref[...]
    e = emb_ref[...]

    zz = jnp.sum(z * z, axis=1, keepdims=True)
    ee = jnp.sum(e * e, axis=1)
    two_ze = 2.0 * jax.lax.dot_general(
        z, e, (((1,), (1,)), ((), ())), preferred_element_type=jnp.float32)
    d = (zz + ee[None, :]) - two_ze

    iota = jax.lax.broadcasted_iota(jnp.int32, (TILE, N_E), 1)
    dmin = jnp.min(d, axis=1, keepdims=True)
    idx = jnp.min(jnp.where(d == dmin, iota, N_E), axis=1)
    idx_ref[...] = idx[:, None]

    one_hot = (iota == idx[:, None]).astype(jnp.float32)
    z_q = jax.lax.dot_general(
        one_hot, e, (((1,), (0,)), ((), ())), preferred_element_type=jnp.float32)
    zq_ref[...] = z + (z_q - z)

    diff = z_q - z
    partial = jnp.sum(diff * diff)

    @pl.when(i == 0)
    def _init():
        loss_ref[0, 0] = partial
        sq = (ee[:, None] + ee[None, :]) - 2.0 * jax.lax.dot_general(
            e, e, (((1,), (1,)), ((), ())), preferred_element_type=jnp.float32)
        sq = jnp.maximum(sq, 0.0)
        dist = jnp.where(sq > 0.0, jnp.sqrt(jnp.where(sq > 0.0, sq, 1.0)), 0.0)
        min_d = jnp.min(dist, axis=1)
        max_d = jnp.max(dist, axis=1)
        uniform_loss = jnp.mean(max_d - min_d)
        qq_ref[0, 0] = UNIFORM_WEIGHT * uniform_loss + LAMBDA_REG * jnp.sum(e * e)

    @pl.when(i != 0)
    def _acc():
        loss_ref[0, 0] += partial


@functools.partial(jax.jit, static_argnames=("interpret",))
def _vq_fused(z_flat, emb, interpret=False):
    n = z_flat.shape[0]
    grid = n // TILE
    zq, idx, loss_sum, qq = pl.pallas_call(
        _vq_body,
        grid=(grid,),
        in_specs=[
            pl.BlockSpec((TILE, E_DIM), lambda i: (i, 0)),
            pl.BlockSpec((N_E, E_DIM), lambda i: (0, 0)),
        ],
        out_specs=[
            pl.BlockSpec((TILE, E_DIM), lambda i: (i, 0)),
            pl.BlockSpec((TILE, 1), lambda i: (i, 0)),
            pl.BlockSpec(memory_space=pltpu.SMEM),
            pl.BlockSpec(memory_space=pltpu.SMEM),
        ],
        out_shape=[
            jax.ShapeDtypeStruct((n, E_DIM), jnp.float32),
            jax.ShapeDtypeStruct((n, 1), jnp.int32),
            jax.ShapeDtypeStruct((1, 1), jnp.float32),
            jax.ShapeDtypeStruct((1, 1), jnp.float32),
        ],
        compiler_params=pltpu.CompilerParams(
            dimension_semantics=("arbitrary",)),
        interpret=interpret,
    )(z_flat, emb)
    return zq, idx, loss_sum, qq


def kernel(input, embedding_weight):
    z = input
    z_flat = z.reshape(-1, E_DIM)
    zq, idx, loss_sum, qq = _vq_fused(z_flat, embedding_weight)
    m = loss_sum[0, 0] / (z_flat.shape[0] * E_DIM)
    loss = m + BETA * m
    return (zq.reshape(z.shape), idx, loss, qq[0, 0])


# speedup vs baseline: 1.5658x; 1.0706x over previous
import functools

import jax
import jax.numpy as jnp
from jax.experimental import pallas as pl
from jax.experimental.pallas import tpu as pltpu

N_E = 1024
E_DIM = 64
BETA = 0.25
LAMBDA_REG = 0.1
UNIFORM_WEIGHT = 0.1

TILE = 4608


def _vq_body(z_ref, emb_ref, zq_ref, idx_ref, loss_ref, qq_ref):
    i = pl.program_id(0)
    z = z_ref[...]
    e = emb_ref[...]

    zz = jnp.sum(z * z, axis=1, keepdims=True)
    ee = jnp.sum(e * e, axis=1)
    two_ze = 2.0 * jax.lax.dot_general(
        z, e, (((1,), (1,)), ((), ())), preferred_element_type=jnp.float32)
    d = (zz + ee[None, :]) - two_ze

    iota = jax.lax.broadcasted_iota(jnp.int32, (TILE, N_E), 1)
    dmin = jnp.min(d, axis=1, keepdims=True)
    eq = d == dmin
    idx = jnp.min(jnp.where(eq, iota, N_E), axis=1)
    idx_ref[...] = idx[:, None]

    one_hot = eq.astype(jnp.float32)
    z_q = jax.lax.dot_general(
        one_hot, e, (((1,), (0,)), ((), ())), preferred_element_type=jnp.float32)
    zq_ref[...] = z + (z_q - z)

    diff = z_q - z
    partial = jnp.sum(diff * diff)

    @pl.when(i == 0)
    def _init():
        loss_ref[0, 0] = partial
        sq = (ee[:, None] + ee[None, :]) - 2.0 * jax.lax.dot_general(
            e, e, (((1,), (1,)), ((), ())), preferred_element_type=jnp.float32)
        sq = jnp.maximum(sq, 0.0)
        dist = jnp.where(sq > 0.0, jnp.sqrt(jnp.where(sq > 0.0, sq, 1.0)), 0.0)
        min_d = jnp.min(dist, axis=1)
        max_d = jnp.max(dist, axis=1)
        uniform_loss = jnp.mean(max_d - min_d)
        qq_ref[0, 0] = UNIFORM_WEIGHT * uniform_loss + LAMBDA_REG * jnp.sum(e * e)

    @pl.when(i != 0)
    def _acc():
        loss_ref[0, 0] += partial


@functools.partial(jax.jit, static_argnames=("interpret",))
def _vq_fused(z_flat, emb, interpret=False):
    n = z_flat.shape[0]
    grid = n // TILE
    zq, idx, loss_sum, qq = pl.pallas_call(
        _vq_body,
        grid=(grid,),
        in_specs=[
            pl.BlockSpec((TILE, E_DIM), lambda i: (i, 0)),
            pl.BlockSpec((N_E, E_DIM), lambda i: (0, 0)),
        ],
        out_specs=[
            pl.BlockSpec((TILE, E_DIM), lambda i: (i, 0)),
            pl.BlockSpec((TILE, 1), lambda i: (i, 0)),
            pl.BlockSpec(memory_space=pltpu.SMEM),
            pl.BlockSpec(memory_space=pltpu.SMEM),
        ],
        out_shape=[
            jax.ShapeDtypeStruct((n, E_DIM), jnp.float32),
            jax.ShapeDtypeStruct((n, 1), jnp.int32),
            jax.ShapeDtypeStruct((1, 1), jnp.float32),
            jax.ShapeDtypeStruct((1, 1), jnp.float32),
        ],
        compiler_params=pltpu.CompilerParams(
            dimension_semantics=("arbitrary",)),
        interpret=interpret,
    )(z_flat, emb)
    return zq, idx, loss_sum, qq


def kernel(input, embedding_weight):
    z = input
    z_flat = z.reshape(-1, E_DIM)
    zq, idx, loss_sum, qq = _vq_fused(z_flat, embedding_weight)
    m = loss_sum[0, 0] / (z_flat.shape[0] * E_DIM)
    loss = m + BETA * m
    return (zq.reshape(z.shape), idx, loss, qq[0, 0])
